# baseline (speedup 1.0000x reference)
import functools

import jax
import jax.numpy as jnp
from jax import lax
from jax.experimental import pallas as pl
from jax.experimental.pallas import tpu as pltpu

_B = 1024
_E = 8
_CH = 16
_P = 784
_XPAD = _P + 2 * 28 + 2
_OFFS = tuple(i * 28 + j for i in range(3) for j in range(3))
_TB = 256
_NT = _B // _TB

_HI = lax.Precision.HIGHEST


def _top2_gates(logits):
    n = logits.shape[-1]
    iota = lax.broadcasted_iota(jnp.int32, logits.shape, 1)
    v1 = jnp.max(logits, axis=1, keepdims=True)
    i1 = jnp.min(jnp.where(logits == v1, iota, n), axis=1, keepdims=True)
    masked = jnp.where(iota == i1, -jnp.inf, logits)
    v2 = jnp.max(masked, axis=1, keepdims=True)
    i2 = jnp.min(jnp.where(masked == v2, iota, n), axis=1, keepdims=True)
    e = jnp.exp(v2 - v1)
    ga = 1.0 / (1.0 + e)
    gb = e / (1.0 + e)
    return jnp.where(iota == i1, ga, 0.0) + jnp.where(iota == i2, gb, 0.0)


def _cv_sq(v, axis):
    m = jnp.mean(v, axis=axis, keepdims=True)
    var = jnp.mean((v - m) ** 2, axis=axis, keepdims=True)
    return var / (m * m + 1e-10)


def _main_kernel(x_ref, wc_ref, bc_ref, wg1_ref, we1_ref,
                 y1_ref, l1_ref, xsh_ref):
    ch = pl.program_id(0)

    @pl.when(ch == 0)
    def _init():
        xv = x_ref[...].astype(jnp.float32)
        for t, off in enumerate(_OFFS):
            xsh_ref[t] = xv[:, off:off + _P]
        y1_ref[...] = jnp.zeros_like(y1_ref)
        l1_ref[...] = jnp.zeros_like(l1_ref)

    conv = bc_ref[ch, 0] + wc_ref[ch, 0] * xsh_ref[0]
    for t in range(1, 9):
        conv = conv + wc_ref[ch, t] * xsh_ref[t]
    hch = jnp.maximum(conv, 0.0)

    hb = hch.astype(jnp.bfloat16)
    y1_ref[...] += jnp.dot(hb, we1_ref[0], preferred_element_type=jnp.float32)
    l1_ref[...] += jnp.dot(hb, wg1_ref[0], preferred_element_type=jnp.float32)


def _epilogue_kernel(y1_ref, l1_ref, be1_ref, wg2_ref, we2_ref, be2_ref,
                     out_ref, g1_ref, g2_ref, loss_ref, stat_ref):
    b = pl.program_id(0)
    g1 = _top2_gates(l1_ref[...])
    g1_ref[...] = g1
    y = y1_ref[...] + be1_ref[...]
    gb = g1.astype(jnp.bfloat16).astype(jnp.float32)
    yb = y.astype(jnp.bfloat16).astype(jnp.float32)
    o1 = gb[:, 0:1] * yb[:, 0:128]
    for e in range(1, _E):
        o1 = o1 + gb[:, e:e + 1] * yb[:, e * 128:(e + 1) * 128]
    o1 = jnp.maximum(o1, 0.0)

    o1b = o1.astype(jnp.bfloat16)
    l2 = jnp.dot(o1b, wg2_ref[...], preferred_element_type=jnp.float32)
    g2 = _top2_gates(l2)
    g2_ref[...] = g2
    g2b = g2.astype(jnp.bfloat16).astype(jnp.float32)
    o2 = jnp.zeros((_TB, 10), jnp.float32)
    for e in range(_E):
        y2 = jnp.dot(o1b, we2_ref[e], preferred_element_type=jnp.float32)
        y2 = (y2 + be2_ref[e:e + 1, :]).astype(jnp.bfloat16).astype(jnp.float32)
        o2 = o2 + g2b[:, e:e + 1] * y2

    m = jnp.max(o2, axis=1, keepdims=True)
    s = o2 - m
    lse = jnp.log(jnp.sum(jnp.exp(s), axis=1, keepdims=True))
    out_ref[...] = s - lse

    part = jnp.concatenate(
        [jnp.sum(g1, axis=0, keepdims=True),
         jnp.sum((g1 > 0.0).astype(jnp.float32), axis=0, keepdims=True),
         jnp.sum(g2, axis=0, keepdims=True),
         jnp.sum((g2 > 0.0).astype(jnp.float32), axis=0, keepdims=True)],
        axis=0)

    @pl.when(b == 0)
    def _():
        stat_ref[...] = part

    @pl.when(b > 0)
    def _():
        stat_ref[...] += part

    @pl.when(b == _NT - 1)
    def _():
        st = stat_ref[...]
        loss = (_cv_sq(st[0:1], 1) + _cv_sq(st[1:2], 1)
                + _cv_sq(st[2:3], 1) + _cv_sq(st[3:4], 1)) * 3e-05
        loss_ref[...] = loss


@functools.partial(jax.jit, static_argnames=())
def kernel(x, Wc, bc, wg1, We1, be1, wg2, We2, be2):
    x2 = jnp.pad(x.reshape(_B, _P).astype(jnp.bfloat16),
                 ((0, 0), (0, _XPAD - _P)))
    wc9 = lax.optimization_barrier(
        Wc.reshape(_CH, 9).astype(jnp.bfloat16)).astype(jnp.float32)
    bc2 = bc.reshape(_CH, 1)
    we1v = We1.reshape(_E, _CH, 26, 26, 128)
    we1p = jnp.pad(we1v, ((0, 0), (0, 0), (0, 2), (0, 2), (0, 0)))
    we1t = we1p.transpose(1, 2, 3, 0, 4).reshape(_CH, _P, _E * 128)
    we1t = we1t.astype(jnp.bfloat16)
    wg1v = wg1.reshape(_CH, 26, 26, _E)
    wg1p = jnp.pad(wg1v, ((0, 0), (0, 2), (0, 2), (0, 0))).reshape(_CH, _P, _E)
    wg1p = wg1p.astype(jnp.bfloat16)
    be1r = be1.reshape(1, _E * 128)
    wg2b = wg2.astype(jnp.bfloat16)
    we2b = We2.astype(jnp.bfloat16)

    y1, l1 = pl.pallas_call(
        _main_kernel,
        grid=(_CH,),
        in_specs=[
            pl.BlockSpec((_B, _XPAD), lambda ch: (0, 0)),
            pl.BlockSpec(memory_space=pltpu.SMEM),
            pl.BlockSpec(memory_space=pltpu.SMEM),
            pl.BlockSpec((1, _P, _E), lambda ch: (ch, 0, 0)),
            pl.BlockSpec((1, _P, _E * 128), lambda ch: (ch, 0, 0)),
        ],
        out_specs=[
            pl.BlockSpec((_B, _E * 128), lambda ch: (0, 0)),
            pl.BlockSpec((_B, _E), lambda ch: (0, 0)),
        ],
        out_shape=[
            jax.ShapeDtypeStruct((_B, _E * 128), jnp.float32),
            jax.ShapeDtypeStruct((_B, _E), jnp.float32),
        ],
        scratch_shapes=[
            pltpu.VMEM((9, _B, _P), jnp.float32),
        ],
    )(x2, wc9, bc2, wg1p, we1t)

    out, g1, g2, loss = pl.pallas_call(
        _epilogue_kernel,
        grid=(_NT,),
        in_specs=[
            pl.BlockSpec((_TB, _E * 128), lambda b: (b, 0)),
            pl.BlockSpec((_TB, _E), lambda b: (b, 0)),
            pl.BlockSpec((1, _E * 128), lambda b: (0, 0)),
            pl.BlockSpec((128, _E), lambda b: (0, 0)),
            pl.BlockSpec((_E, 128, 10), lambda b: (0, 0, 0)),
            pl.BlockSpec((_E, 10), lambda b: (0, 0)),
        ],
        out_specs=[
            pl.BlockSpec((_TB, 10), lambda b: (b, 0)),
            pl.BlockSpec((_TB, _E), lambda b: (b, 0)),
            pl.BlockSpec((_TB, _E), lambda b: (b, 0)),
            pl.BlockSpec((1, 1), lambda b: (0, 0)),
        ],
        out_shape=[
            jax.ShapeDtypeStruct((_B, 10), jnp.float32),
            jax.ShapeDtypeStruct((_B, _E), jnp.float32),
            jax.ShapeDtypeStruct((_B, _E), jnp.float32),
            jax.ShapeDtypeStruct((1, 1), jnp.float32),
        ],
        scratch_shapes=[
            pltpu.VMEM((4, _E), jnp.float32),
        ],
    )(y1, l1, be1r, wg2b, we2b, be2)
    return out, g1, g2, loss.reshape(())
